# baseline (speedup 1.0000x reference)
import functools

import numpy as np
import jax
import jax.numpy as jnp
from jax import lax
from jax.experimental import pallas as pl
from jax.experimental.pallas import tpu as pltpu
from jax.experimental.pallas import tpu_sc as plsc

_N = 10000
_E = 320000
_D = 128
_DW = 136
_H = 8
_DH = 16

_NC = 2
_NS = 16
_NW = _NC * _NS
_EPW = _E // _NW
_BE = 40
_NBLK = _EPW // _BE
_NPAIR = _NBLK // 2
_RPT = 624
_TAIL = _N - _NS * _RPT

_f32 = jnp.float32
_i32 = jnp.int32


def _proj_body(x_ref, wkv_ref, wq_ref, kv_ref, q_ref):
    xb = x_ref[...]
    kv_ref[...] = jnp.dot(xb, wkv_ref[...], preferred_element_type=_f32)
    q_ref[...] = jnp.dot(xb, wq_ref[...], preferred_element_type=_f32) * 0.25


_proj = pl.pallas_call(
    _proj_body,
    grid=(10,),
    in_specs=[
        pl.BlockSpec((1000, _D), lambda i: (i, 0)),
        pl.BlockSpec((_D, 2 * _D), lambda i: (0, 0)),
        pl.BlockSpec((_D, _D), lambda i: (0, 0)),
    ],
    out_specs=[
        pl.BlockSpec((1000, 2 * _D), lambda i: (i, 0)),
        pl.BlockSpec((1000, _D), lambda i: (i, 0)),
    ],
    out_shape=[
        jax.ShapeDtypeStruct((_N, 2 * _D), _f32),
        jax.ShapeDtypeStruct((_N, _D), _f32),
    ],
)


def _ehm_body(ea_ref, we_ref, out_ref):
    out_ref[...] = jnp.dot(ea_ref[...], we_ref[...],
                           preferred_element_type=_f32)


_ehm = pl.pallas_call(
    _ehm_body,
    grid=(160,),
    in_specs=[
        pl.BlockSpec((2000, _DH), lambda i: (i, 0)),
        pl.BlockSpec((_DH, _DW), lambda i: (0, 0)),
    ],
    out_specs=pl.BlockSpec((2000, _DW), lambda i: (i, 0)),
    out_shape=jax.ShapeDtypeStruct((_E, _DW), _f32),
)


def _sc_body(kv_hbm, q_hbm, eh_hbm, pidx_hbm,
             wv_out,
             pidx0, kv0, q0, eh0,
             pidx1, kv1, q1, eh1,
             wv_sh, semi, semg0, semg1, sems):
    c = lax.axis_index("c")
    s = lax.axis_index("s")
    wid = s * _NC + c
    b_w0 = wid * _NBLK

    iota16 = lax.iota(_i32, 16)
    zeros16 = jnp.zeros((16,), _f32)
    ones16 = jnp.full((16,), 1, _i32)
    fifteen16 = jnp.full((16,), 15, _i32)
    v128 = jnp.full((16,), _D, _i32)

    rz = iota16 >> 3
    cz = (iota16 & 7) + _D

    def _zero_eh(r, carry):
        for cc in range(8):
            eh0[r, pl.ds(cc * 16, 16)] = zeros16
        return carry

    lax.fori_loop(0, _BE, _zero_eh, 0)
    for k in range(_BE // 2):
        plsc.store_scatter(eh0, [rz + 2 * k, cz], zeros16)

    start = s * _RPT
    for i in range(15):
        pltpu.sync_copy(eh0, wv_sh.at[pl.ds(start + i * _BE, _BE), :])
    pltpu.sync_copy(eh0.at[pl.ds(0, 24), :],
                    wv_sh.at[pl.ds(start + 600, 24), :])

    @pl.when(s == _NS - 1)
    def _zero_tail():
        pltpu.sync_copy(eh0.at[pl.ds(0, _TAIL), :],
                        wv_sh.at[pl.ds(_NS * _RPT, _TAIL), :])

    plsc.subcore_barrier()

    def _load_idx(bid, pidx):
        pltpu.async_copy(pidx_hbm.at[bid], pidx, semi).wait()

    def _issue_gathers(bid, pidx, kv, q, eh, semg):
        pltpu.async_copy(kv_hbm.at[pidx.at[0]], kv, semg)
        pltpu.async_copy(q_hbm.at[pidx.at[1]], q, semg)
        pltpu.async_copy(eh_hbm.at[pl.ds(bid * _BE, _BE), :], eh, semg)

    def _drain_gathers(bid, pidx, kv, q, eh, semg):
        pltpu.make_async_copy(kv_hbm.at[pidx.at[0]], kv, semg).wait()
        pltpu.make_async_copy(q_hbm.at[pidx.at[1]], q, semg).wait()
        pltpu.make_async_copy(
            eh_hbm.at[pl.ds(bid * _BE, _BE), :], eh, semg).wait()

    def _compute(kv, q, eh):
        def grp(g, carry):
            el = g * 16
            rows = iota16 + el
            mask = rows < _BE

            def head(h, counters):
                cb, hcol = counters
                rot = iota16
                acc = zeros16
                for d in range(_DH):
                    cv = cb | rot
                    kc = plsc.load_gather(kv, [rows, cv], mask=mask)
                    qc = plsc.load_gather(q, [rows, cv], mask=mask)
                    ec = plsc.load_gather(eh, [rows, cv], mask=mask)
                    acc = acc + kc * qc * ec
                    if d < _DH - 1:
                        rot = (rot + ones16) & fifteen16
                sv = jnp.exp(jnp.clip(acc, -5.0, 5.0))
                plsc.store_scatter(eh, [rows, hcol], sv, mask=mask)
                rot = iota16
                for d in range(_DH):
                    cv = cb | rot
                    vc = plsc.load_gather(kv, [rows, cv + v128], mask=mask)
                    plsc.store_scatter(eh, [rows, cv], vc * sv, mask=mask)
                    if d < _DH - 1:
                        rot = (rot + ones16) & fifteen16
                return (cb + _DH, hcol + ones16)

            lax.fori_loop(0, _H, head,
                          (jnp.zeros((16,), _i32), v128))
            return carry

        lax.fori_loop(0, 3, grp, 0)

    def _scatter_sync(eh, pidx):
        pltpu.async_copy(eh, wv_sh.at[pidx.at[1]], sems, add=True).wait()

    _load_idx(b_w0, pidx0)
    _issue_gathers(b_w0, pidx0, kv0, q0, eh0, semg0)

    def pair(i, carry):
        bid0 = b_w0 + i * 2
        bid1 = bid0 + 1
        bid2 = bid0 + 2
        _load_idx(bid1, pidx1)
        _issue_gathers(bid1, pidx1, kv1, q1, eh1, semg1)
        _drain_gathers(bid0, pidx0, kv0, q0, eh0, semg0)
        _compute(kv0, q0, eh0)
        _scatter_sync(eh0, pidx0)
        @pl.when(i < _NPAIR - 1)
        def _refill():
            _load_idx(bid2, pidx0)
            _issue_gathers(bid2, pidx0, kv0, q0, eh0, semg0)

        _drain_gathers(bid1, pidx1, kv1, q1, eh1, semg1)
        _compute(kv1, q1, eh1)
        _scatter_sync(eh1, pidx1)
        return carry

    lax.fori_loop(0, _NPAIR, pair, 0)

    plsc.subcore_barrier()
    pltpu.sync_copy(wv_sh.at[pl.ds(start, _RPT), :],
                    wv_out.at[c, pl.ds(start, _RPT), :])

    @pl.when(s == _NS - 1)
    def _copy_tail():
        pltpu.sync_copy(wv_sh.at[pl.ds(_NS * _RPT, _TAIL), :],
                        wv_out.at[c, pl.ds(_NS * _RPT, _TAIL), :])


_sc = functools.partial(
    pl.kernel,
    mesh=plsc.VectorSubcoreMesh(core_axis_name="c", subcore_axis_name="s"),
    compiler_params=pltpu.CompilerParams(
        use_tc_tiling_on_sc=False, needs_layout_passes=False),
    out_type=[
        jax.ShapeDtypeStruct((_NC, _N, _DW), _f32),
    ],
    scratch_types=[
        pltpu.VMEM((2, _BE), _i32),
        pltpu.VMEM((_BE, 2 * _D), _f32),
        pltpu.VMEM((_BE, _D), _f32),
        pltpu.VMEM((_BE, _DW), _f32),
        pltpu.VMEM((2, _BE), _i32),
        pltpu.VMEM((_BE, 2 * _D), _f32),
        pltpu.VMEM((_BE, _D), _f32),
        pltpu.VMEM((_BE, _DW), _f32),
        pltpu.VMEM_SHARED((_N, _DW), _f32),
        pltpu.SemaphoreType.DMA,
        pltpu.SemaphoreType.DMA,
        pltpu.SemaphoreType.DMA,
        pltpu.SemaphoreType.DMA,
    ],
)(_sc_body)


def _fin_body(wv_ref, r_ref, o_ref):
    w0 = wv_ref[0]
    w1 = wv_ref[1]
    zs = w0[:, _D:] + w1[:, _D:]
    recip = 1.0 / (zs + 1e-6)
    zb = jnp.dot(recip, r_ref[...], preferred_element_type=_f32)
    o_ref[...] = (w0[:, :_D] + w1[:, :_D]) * zb


_fin = pl.pallas_call(
    _fin_body,
    grid=(10,),
    in_specs=[
        pl.BlockSpec((_NC, 1000, _DW), lambda i: (0, i, 0)),
        pl.BlockSpec((_H, _D), lambda i: (0, 0)),
    ],
    out_specs=pl.BlockSpec((1000, _D), lambda i: (i, 0)),
    out_shape=jax.ShapeDtypeStruct((_N, _D), _f32),
)

_RNP = np.kron(np.eye(_H), np.ones((1, _DH))).astype(np.float32)


def kernel(x, edge_index, edge_attr, WQ, WK, WE, WV):
    wkv = jnp.concatenate([WK, WV], axis=1)
    kv, q = _proj(x, wkv, WQ)
    wep = jnp.concatenate([WE, jnp.zeros((_DH, _DW - _D), _f32)], axis=1)
    eh = _ehm(edge_attr, wep)
    pidx = jnp.swapaxes(edge_index.reshape(2, _E // _BE, _BE), 0, 1)
    wv_p, = _sc(kv, q, eh, pidx)
    return _fin(wv_p, jnp.asarray(_RNP))


# speedup vs baseline: 1.3398x; 1.3398x over previous
import functools

import numpy as np
import jax
import jax.numpy as jnp
from jax import lax
from jax.experimental import pallas as pl
from jax.experimental.pallas import tpu as pltpu
from jax.experimental.pallas import tpu_sc as plsc

_N = 10000
_E = 320000
_D = 128
_H = 8
_DH = 16

_NC = 2
_NS = 16
_NW = _NC * _NS
_EPW = _E // _NW
_BE = 40
_NBLK = _EPW // _BE
_NPAIR = _NBLK // 2
_RPT = 624
_TAIL = _N - _NS * _RPT

_f32 = jnp.float32
_i32 = jnp.int32


def _proj_body(x_ref, wkv_ref, wq_ref, kv_ref, q_ref):
    xb = x_ref[...]
    kv_ref[...] = jnp.dot(xb, wkv_ref[...], preferred_element_type=_f32)
    q_ref[...] = jnp.dot(xb, wq_ref[...], preferred_element_type=_f32) * 0.25


_proj = pl.pallas_call(
    _proj_body,
    grid=(10,),
    in_specs=[
        pl.BlockSpec((1000, _D), lambda i: (i, 0)),
        pl.BlockSpec((_D, 2 * _D), lambda i: (0, 0)),
        pl.BlockSpec((_D, _D), lambda i: (0, 0)),
    ],
    out_specs=[
        pl.BlockSpec((1000, 2 * _D), lambda i: (i, 0)),
        pl.BlockSpec((1000, _D), lambda i: (i, 0)),
    ],
    out_shape=[
        jax.ShapeDtypeStruct((_N, 2 * _D), _f32),
        jax.ShapeDtypeStruct((_N, _D), _f32),
    ],
)


def _ehm_body(ea_ref, we_ref, out_ref):
    out_ref[...] = jnp.dot(ea_ref[...], we_ref[...],
                           preferred_element_type=_f32)


_ehm = pl.pallas_call(
    _ehm_body,
    grid=(160,),
    in_specs=[
        pl.BlockSpec((2000, _DH), lambda i: (i, 0)),
        pl.BlockSpec((_DH, _D), lambda i: (0, 0)),
    ],
    out_specs=pl.BlockSpec((2000, _D), lambda i: (i, 0)),
    out_shape=jax.ShapeDtypeStruct((_E, _D), _f32),
)


def _sc_body(kv_hbm, q_hbm, eh_hbm, pidx_hbm,
             wv_out, z_out,
             pidx0, kv0, q0, eh0, zr0,
             pidx1, kv1, q1, eh1, zr1,
             wv_sh, z_sh, semi, semg0, semg1, sems):
    c = lax.axis_index("c")
    s = lax.axis_index("s")
    wid = s * _NC + c
    b_w0 = wid * _NBLK

    iota16 = lax.iota(_i32, 16)
    zeros16 = jnp.zeros((16,), _f32)
    ones16 = jnp.full((16,), 1, _i32)
    fifteen16 = jnp.full((16,), 15, _i32)
    v128 = jnp.full((16,), _D, _i32)

    def _zero_eh(r, carry):
        for cc in range(8):
            eh0[r, pl.ds(cc * 16, 16)] = zeros16
        return carry

    lax.fori_loop(0, _BE, _zero_eh, 0)
    rz = iota16 >> 3
    cz = iota16 & 7
    for k in range(_BE // 2):
        plsc.store_scatter(zr0, [rz + 2 * k, cz], zeros16)

    start = s * _RPT
    for i in range(15):
        pltpu.sync_copy(eh0, wv_sh.at[pl.ds(start + i * _BE, _BE), :])
        pltpu.sync_copy(zr0, z_sh.at[pl.ds(start + i * _BE, _BE), :])
    pltpu.sync_copy(eh0.at[pl.ds(0, 24), :],
                    wv_sh.at[pl.ds(start + 600, 24), :])
    pltpu.sync_copy(zr0.at[pl.ds(0, 24), :],
                    z_sh.at[pl.ds(start + 600, 24), :])

    @pl.when(s == _NS - 1)
    def _zero_tail():
        pltpu.sync_copy(eh0.at[pl.ds(0, _TAIL), :],
                        wv_sh.at[pl.ds(_NS * _RPT, _TAIL), :])
        pltpu.sync_copy(zr0.at[pl.ds(0, _TAIL), :],
                        z_sh.at[pl.ds(_NS * _RPT, _TAIL), :])

    plsc.subcore_barrier()

    def _load_idx(bid, pidx):
        pltpu.async_copy(pidx_hbm.at[bid], pidx, semi).wait()

    def _issue_gathers(bid, pidx, kv, q, eh, semg):
        pltpu.async_copy(kv_hbm.at[pidx.at[0]], kv, semg)
        pltpu.async_copy(q_hbm.at[pidx.at[1]], q, semg)
        pltpu.async_copy(eh_hbm.at[pl.ds(bid * _BE, _BE), :], eh, semg)

    def _drain_gathers(bid, pidx, kv, q, eh, semg):
        pltpu.make_async_copy(kv_hbm.at[pidx.at[0]], kv, semg).wait()
        pltpu.make_async_copy(q_hbm.at[pidx.at[1]], q, semg).wait()
        pltpu.make_async_copy(
            eh_hbm.at[pl.ds(bid * _BE, _BE), :], eh, semg).wait()

    def _compute(kv, q, eh, zr):
        def grp(g, carry):
            el = g * 16
            rows = iota16 + el
            mask = rows < _BE

            def head(h, counters):
                cb, hcol = counters
                rot = iota16
                acc = zeros16
                for d in range(_DH):
                    cv = cb | rot
                    kc = plsc.load_gather(kv, [rows, cv], mask=mask)
                    qc = plsc.load_gather(q, [rows, cv], mask=mask)
                    ec = plsc.load_gather(eh, [rows, cv], mask=mask)
                    acc = acc + kc * qc * ec
                    if d < _DH - 1:
                        rot = (rot + ones16) & fifteen16
                sv = jnp.exp(jnp.clip(acc, -5.0, 5.0))
                plsc.store_scatter(zr, [rows, hcol], sv, mask=mask)
                rot = iota16
                for d in range(_DH):
                    cv = cb | rot
                    vc = plsc.load_gather(kv, [rows, cv + v128], mask=mask)
                    plsc.store_scatter(eh, [rows, cv], vc * sv, mask=mask)
                    if d < _DH - 1:
                        rot = (rot + ones16) & fifteen16
                return (cb + _DH, hcol + ones16)

            lax.fori_loop(0, _H, head,
                          (jnp.zeros((16,), _i32), jnp.zeros((16,), _i32)))
            return carry

        lax.fori_loop(0, 3, grp, 0)

    def _scatter_sync(eh, zr, pidx):
        c1 = pltpu.async_copy(eh, wv_sh.at[pidx.at[1]], sems, add=True)
        c2 = pltpu.async_copy(zr, z_sh.at[pidx.at[1]], sems, add=True)
        c1.wait()
        c2.wait()

    _load_idx(b_w0, pidx0)
    _issue_gathers(b_w0, pidx0, kv0, q0, eh0, semg0)

    def pair(i, carry):
        bid0 = b_w0 + i * 2
        bid1 = bid0 + 1
        bid2 = bid0 + 2
        _load_idx(bid1, pidx1)
        _issue_gathers(bid1, pidx1, kv1, q1, eh1, semg1)
        _drain_gathers(bid0, pidx0, kv0, q0, eh0, semg0)
        _compute(kv0, q0, eh0, zr0)
        _scatter_sync(eh0, zr0, pidx0)
        @pl.when(i < _NPAIR - 1)
        def _refill():
            _load_idx(bid2, pidx0)
            _issue_gathers(bid2, pidx0, kv0, q0, eh0, semg0)

        _drain_gathers(bid1, pidx1, kv1, q1, eh1, semg1)
        _compute(kv1, q1, eh1, zr1)
        _scatter_sync(eh1, zr1, pidx1)
        return carry

    lax.fori_loop(0, _NPAIR, pair, 0)

    plsc.subcore_barrier()
    pltpu.sync_copy(wv_sh.at[pl.ds(start, _RPT), :],
                    wv_out.at[c, pl.ds(start, _RPT), :])
    pltpu.sync_copy(z_sh.at[pl.ds(start, _RPT), :],
                    z_out.at[c, pl.ds(start, _RPT), :])

    @pl.when(s == _NS - 1)
    def _copy_tail():
        pltpu.sync_copy(wv_sh.at[pl.ds(_NS * _RPT, _TAIL), :],
                        wv_out.at[c, pl.ds(_NS * _RPT, _TAIL), :])
        pltpu.sync_copy(z_sh.at[pl.ds(_NS * _RPT, _TAIL), :],
                        z_out.at[c, pl.ds(_NS * _RPT, _TAIL), :])


_sc = functools.partial(
    pl.kernel,
    mesh=plsc.VectorSubcoreMesh(core_axis_name="c", subcore_axis_name="s"),
    compiler_params=pltpu.CompilerParams(
        use_tc_tiling_on_sc=False, needs_layout_passes=False),
    out_type=[
        jax.ShapeDtypeStruct((_NC, _N, _D), _f32),
        jax.ShapeDtypeStruct((_NC, _N, _H), _f32),
    ],
    scratch_types=[
        pltpu.VMEM((2, _BE), _i32),
        pltpu.VMEM((_BE, 2 * _D), _f32),
        pltpu.VMEM((_BE, _D), _f32),
        pltpu.VMEM((_BE, _D), _f32),
        pltpu.VMEM((_BE, _H), _f32),
        pltpu.VMEM((2, _BE), _i32),
        pltpu.VMEM((_BE, 2 * _D), _f32),
        pltpu.VMEM((_BE, _D), _f32),
        pltpu.VMEM((_BE, _D), _f32),
        pltpu.VMEM((_BE, _H), _f32),
        pltpu.VMEM_SHARED((_N, _D), _f32),
        pltpu.VMEM_SHARED((_N, _H), _f32),
        pltpu.SemaphoreType.DMA,
        pltpu.SemaphoreType.DMA,
        pltpu.SemaphoreType.DMA,
        pltpu.SemaphoreType.DMA,
    ],
)(_sc_body)


def _fin_body(wv_ref, z_ref, r_ref, o_ref):
    zs = z_ref[0] + z_ref[1]
    recip = 1.0 / (zs + 1e-6)
    zb = jnp.dot(recip, r_ref[...], preferred_element_type=_f32)
    o_ref[...] = (wv_ref[0] + wv_ref[1]) * zb


_fin = pl.pallas_call(
    _fin_body,
    grid=(10,),
    in_specs=[
        pl.BlockSpec((_NC, 1000, _D), lambda i: (0, i, 0)),
        pl.BlockSpec((_NC, 1000, _H), lambda i: (0, i, 0)),
        pl.BlockSpec((_H, _D), lambda i: (0, 0)),
    ],
    out_specs=pl.BlockSpec((1000, _D), lambda i: (i, 0)),
    out_shape=jax.ShapeDtypeStruct((_N, _D), _f32),
)

_RNP = np.kron(np.eye(_H), np.ones((1, _DH))).astype(np.float32)


def kernel(x, edge_index, edge_attr, WQ, WK, WE, WV):
    wkv = jnp.concatenate([WK, WV], axis=1)
    kv, q = _proj(x, wkv, WQ)
    eh = _ehm(edge_attr, WE)
    pidx = jnp.swapaxes(edge_index.reshape(2, _E // _BE, _BE), 0, 1)
    wv_p, z_p = _sc(kv, q, eh, pidx)
    return _fin(wv_p, z_p, jnp.asarray(_RNP))


# speedup vs baseline: 1.4520x; 1.0838x over previous
import functools

import numpy as np
import jax
import jax.numpy as jnp
from jax import lax
from jax.experimental import pallas as pl
from jax.experimental.pallas import tpu as pltpu
from jax.experimental.pallas import tpu_sc as plsc

_N = 10000
_E = 320000
_D = 128
_H = 8
_DH = 16

_NC = 2
_NS = 16
_NW = _NC * _NS
_EPW = _E // _NW
_BE = 40
_NBLK = _EPW // _BE
_NPAIR = _NBLK // 2
_RPT = 624
_TAIL = _N - _NS * _RPT

_f32 = jnp.float32
_i32 = jnp.int32


def _proj_body(x_ref, wkv_ref, wq_ref, kv_ref, q_ref):
    xb = x_ref[...]
    kv_ref[...] = jnp.dot(xb, wkv_ref[...], preferred_element_type=_f32)
    q_ref[...] = jnp.dot(xb, wq_ref[...], preferred_element_type=_f32) * 0.25


_proj = pl.pallas_call(
    _proj_body,
    grid=(10,),
    in_specs=[
        pl.BlockSpec((1000, _D), lambda i: (i, 0)),
        pl.BlockSpec((_D, 2 * _D), lambda i: (0, 0)),
        pl.BlockSpec((_D, _D), lambda i: (0, 0)),
    ],
    out_specs=[
        pl.BlockSpec((1000, 2 * _D), lambda i: (i, 0)),
        pl.BlockSpec((1000, _D), lambda i: (i, 0)),
    ],
    out_shape=[
        jax.ShapeDtypeStruct((_N, 2 * _D), _f32),
        jax.ShapeDtypeStruct((_N, _D), _f32),
    ],
)


def _ehm_body(ea_ref, we_ref, out_ref):
    out_ref[...] = jnp.dot(ea_ref[...], we_ref[...],
                           preferred_element_type=_f32)


_ehm = pl.pallas_call(
    _ehm_body,
    grid=(160,),
    in_specs=[
        pl.BlockSpec((2000, _DH), lambda i: (i, 0)),
        pl.BlockSpec((_DH, _D), lambda i: (0, 0)),
    ],
    out_specs=pl.BlockSpec((2000, _D), lambda i: (i, 0)),
    out_shape=jax.ShapeDtypeStruct((_E, _D), _f32),
)


def _sc_body(kv_hbm, q_hbm, eh_hbm, pidx_hbm,
             wv_out, z_out,
             pidx0, kv0, q0, eh0, zr0, didxs0,
             pidx1, kv1, q1, eh1, zr1, didxs1,
             wv_sh, z_sh, semi, semg0, semg1, sems):
    c = lax.axis_index("c")
    s = lax.axis_index("s")
    wid = s * _NC + c
    b_w0 = wid * _NBLK

    iota16 = lax.iota(_i32, 16)
    zeros16 = jnp.zeros((16,), _f32)
    ones16 = jnp.full((16,), 1, _i32)
    fifteen16 = jnp.full((16,), 15, _i32)
    v128 = jnp.full((16,), _D, _i32)

    def _zero_eh(r, carry):
        for cc in range(8):
            eh0[r, pl.ds(cc * 16, 16)] = zeros16
        return carry

    lax.fori_loop(0, _BE, _zero_eh, 0)
    rz = iota16 >> 3
    cz = iota16 & 7
    for k in range(_BE // 2):
        plsc.store_scatter(zr0, [rz + 2 * k, cz], zeros16)

    start = s * _RPT
    for i in range(15):
        pltpu.sync_copy(eh0, wv_sh.at[pl.ds(start + i * _BE, _BE), :])
        pltpu.sync_copy(zr0, z_sh.at[pl.ds(start + i * _BE, _BE), :])
    pltpu.sync_copy(eh0.at[pl.ds(0, 24), :],
                    wv_sh.at[pl.ds(start + 600, 24), :])
    pltpu.sync_copy(zr0.at[pl.ds(0, 24), :],
                    z_sh.at[pl.ds(start + 600, 24), :])

    @pl.when(s == _NS - 1)
    def _zero_tail():
        pltpu.sync_copy(eh0.at[pl.ds(0, _TAIL), :],
                        wv_sh.at[pl.ds(_NS * _RPT, _TAIL), :])
        pltpu.sync_copy(zr0.at[pl.ds(0, _TAIL), :],
                        z_sh.at[pl.ds(_NS * _RPT, _TAIL), :])

    plsc.subcore_barrier()

    def _load_idx(bid, pidx):
        pltpu.async_copy(pidx_hbm.at[bid], pidx, semi).wait()

    def _issue_gathers(bid, pidx, kv, q, eh, semg):
        pltpu.async_copy(kv_hbm.at[pidx.at[0]], kv, semg)
        pltpu.async_copy(q_hbm.at[pidx.at[1]], q, semg)
        pltpu.async_copy(eh_hbm.at[pl.ds(bid * _BE, _BE), :], eh, semg)

    def _drain_gathers(bid, pidx, kv, q, eh, semg):
        pltpu.make_async_copy(kv_hbm.at[pidx.at[0]], kv, semg).wait()
        pltpu.make_async_copy(q_hbm.at[pidx.at[1]], q, semg).wait()
        pltpu.make_async_copy(
            eh_hbm.at[pl.ds(bid * _BE, _BE), :], eh, semg).wait()

    def _compute(kv, q, eh, zr):
        def grp(g, carry):
            el = g * 16
            rows = iota16 + el
            mask = rows < _BE

            def head(h, counters):
                cb, hcol = counters
                rot = iota16
                acc = zeros16
                for d in range(_DH):
                    cv = cb | rot
                    kc = plsc.load_gather(kv, [rows, cv], mask=mask)
                    qc = plsc.load_gather(q, [rows, cv], mask=mask)
                    ec = plsc.load_gather(eh, [rows, cv], mask=mask)
                    acc = acc + kc * qc * ec
                    if d < _DH - 1:
                        rot = (rot + ones16) & fifteen16
                sv = jnp.exp(jnp.clip(acc, -5.0, 5.0))
                plsc.store_scatter(zr, [rows, hcol], sv, mask=mask)
                rot = iota16
                for d in range(_DH):
                    cv = cb | rot
                    vc = plsc.load_gather(kv, [rows, cv + v128], mask=mask)
                    plsc.store_scatter(eh, [rows, cv], vc * sv, mask=mask)
                    if d < _DH - 1:
                        rot = (rot + ones16) & fifteen16
                return (cb + _DH, hcol + ones16)

            lax.fori_loop(0, _H, head,
                          (jnp.zeros((16,), _i32), jnp.zeros((16,), _i32)))
            return carry

        lax.fori_loop(0, 3, grp, 0)

    def _save_didx(pidx, didxs):
        didxs[pl.ds(0, 16)] = pidx[1, pl.ds(0, 16)]
        didxs[pl.ds(16, 16)] = pidx[1, pl.ds(16, 16)]
        didxs[pl.ds(24, 16)] = pidx[1, pl.ds(24, 16)]

    def _scatter_sync(eh, zr, didxs):
        c1 = pltpu.async_copy(eh, wv_sh.at[didxs], sems, add=True)
        c2 = pltpu.async_copy(zr, z_sh.at[didxs], sems, add=True)
        c1.wait()
        c2.wait()

    _load_idx(b_w0, pidx0)
    _issue_gathers(b_w0, pidx0, kv0, q0, eh0, semg0)
    _load_idx(b_w0 + 1, pidx1)
    _issue_gathers(b_w0 + 1, pidx1, kv1, q1, eh1, semg1)

    def pair(i, carry):
        bid0 = b_w0 + i * 2
        bid1 = bid0 + 1
        bid2 = bid0 + 2
        bid3 = bid0 + 3
        not_last = i < _NPAIR - 1
        _save_didx(pidx0, didxs0)
        _drain_gathers(bid0, pidx0, kv0, q0, eh0, semg0)

        @pl.when(not_last)
        def _pref0():
            pltpu.async_copy(pidx_hbm.at[bid2], pidx0, semi)

        _compute(kv0, q0, eh0, zr0)
        _scatter_sync(eh0, zr0, didxs0)

        @pl.when(not_last)
        def _refill0():
            pltpu.make_async_copy(pidx_hbm.at[bid2], pidx0, semi).wait()
            _issue_gathers(bid2, pidx0, kv0, q0, eh0, semg0)

        _save_didx(pidx1, didxs1)
        _drain_gathers(bid1, pidx1, kv1, q1, eh1, semg1)

        @pl.when(not_last)
        def _pref1():
            pltpu.async_copy(pidx_hbm.at[bid3], pidx1, semi)

        _compute(kv1, q1, eh1, zr1)
        _scatter_sync(eh1, zr1, didxs1)

        @pl.when(not_last)
        def _refill1():
            pltpu.make_async_copy(pidx_hbm.at[bid3], pidx1, semi).wait()
            _issue_gathers(bid3, pidx1, kv1, q1, eh1, semg1)

        return carry

    lax.fori_loop(0, _NPAIR, pair, 0)

    plsc.subcore_barrier()
    pltpu.sync_copy(wv_sh.at[pl.ds(start, _RPT), :],
                    wv_out.at[c, pl.ds(start, _RPT), :])
    pltpu.sync_copy(z_sh.at[pl.ds(start, _RPT), :],
                    z_out.at[c, pl.ds(start, _RPT), :])

    @pl.when(s == _NS - 1)
    def _copy_tail():
        pltpu.sync_copy(wv_sh.at[pl.ds(_NS * _RPT, _TAIL), :],
                        wv_out.at[c, pl.ds(_NS * _RPT, _TAIL), :])
        pltpu.sync_copy(z_sh.at[pl.ds(_NS * _RPT, _TAIL), :],
                        z_out.at[c, pl.ds(_NS * _RPT, _TAIL), :])


_sc = functools.partial(
    pl.kernel,
    mesh=plsc.VectorSubcoreMesh(core_axis_name="c", subcore_axis_name="s"),
    compiler_params=pltpu.CompilerParams(
        use_tc_tiling_on_sc=False, needs_layout_passes=False),
    out_type=[
        jax.ShapeDtypeStruct((_NC, _N, _D), _f32),
        jax.ShapeDtypeStruct((_NC, _N, _H), _f32),
    ],
    scratch_types=[
        pltpu.VMEM((2, _BE), _i32),
        pltpu.VMEM((_BE, 2 * _D), _f32),
        pltpu.VMEM((_BE, _D), _f32),
        pltpu.VMEM((_BE, _D), _f32),
        pltpu.VMEM((_BE, _H), _f32),
        pltpu.VMEM((_BE,), _i32),
        pltpu.VMEM((2, _BE), _i32),
        pltpu.VMEM((_BE, 2 * _D), _f32),
        pltpu.VMEM((_BE, _D), _f32),
        pltpu.VMEM((_BE, _D), _f32),
        pltpu.VMEM((_BE, _H), _f32),
        pltpu.VMEM((_BE,), _i32),
        pltpu.VMEM_SHARED((_N, _D), _f32),
        pltpu.VMEM_SHARED((_N, _H), _f32),
        pltpu.SemaphoreType.DMA,
        pltpu.SemaphoreType.DMA,
        pltpu.SemaphoreType.DMA,
        pltpu.SemaphoreType.DMA,
    ],
)(_sc_body)


def _fin_body(wv_ref, z_ref, r_ref, o_ref):
    zs = z_ref[0] + z_ref[1]
    recip = 1.0 / (zs + 1e-6)
    zb = jnp.dot(recip, r_ref[...], preferred_element_type=_f32)
    o_ref[...] = (wv_ref[0] + wv_ref[1]) * zb


_fin = pl.pallas_call(
    _fin_body,
    grid=(10,),
    in_specs=[
        pl.BlockSpec((_NC, 1000, _D), lambda i: (0, i, 0)),
        pl.BlockSpec((_NC, 1000, _H), lambda i: (0, i, 0)),
        pl.BlockSpec((_H, _D), lambda i: (0, 0)),
    ],
    out_specs=pl.BlockSpec((1000, _D), lambda i: (i, 0)),
    out_shape=jax.ShapeDtypeStruct((_N, _D), _f32),
)

_RNP = np.kron(np.eye(_H), np.ones((1, _DH))).astype(np.float32)


def kernel(x, edge_index, edge_attr, WQ, WK, WE, WV):
    wkv = jnp.concatenate([WK, WV], axis=1)
    kv, q = _proj(x, wkv, WQ)
    eh = _ehm(edge_attr, WE)
    pidx = jnp.swapaxes(edge_index.reshape(2, _E // _BE, _BE), 0, 1)
    wv_p, z_p = _sc(kv, q, eh, pidx)
    return _fin(wv_p, z_p, jnp.asarray(_RNP))
